# baseline (speedup 1.0000x reference)
import math

import jax
import jax.numpy as jnp
from jax import lax
from jax.experimental import pallas as pl
from jax.experimental.pallas import tpu as pltpu
from jax.experimental.pallas import tpu_sc as plsc

_SIZE = 100000
_N = 2048
_SMOOTHING = 0.1
_CONF = 1.0 - _SMOOTHING
_S = _SMOOTHING / (_SIZE - 2)
_C0 = (_SIZE - 2) * _S * math.log(_S) + _CONF * math.log(_CONF)

_NC = 2
_NW = 32
_GR = 16
_NG = _N // _NW // _GR
_CW = 2048
_NFULL = 48
_TAILW = 1664
_SC_COLS = _NFULL * _CW + _TAILW
_L = 16


def _row_sum(x_v, b, r, nvec):
    def step(k, s, r=r, b=b):
        base = k * 8 * _L
        t0 = x_v[b, r, pl.ds(base, _L)] + x_v[b, r, pl.ds(base + _L, _L)]
        t1 = (x_v[b, r, pl.ds(base + 2 * _L, _L)]
              + x_v[b, r, pl.ds(base + 3 * _L, _L)])
        t2 = (x_v[b, r, pl.ds(base + 4 * _L, _L)]
              + x_v[b, r, pl.ds(base + 5 * _L, _L)])
        t3 = (x_v[b, r, pl.ds(base + 6 * _L, _L)]
              + x_v[b, r, pl.ds(base + 7 * _L, _L)])
        return s + ((t0 + t1) + (t2 + t3))
    return lax.fori_loop(0, nvec // 8, step, jnp.zeros((_L,), jnp.float32))


def _sc_body(x_hbm, tgt_hbm, out_hbm, x_v, t_v, acc_v, hit_v, stage_v,
             sem0, sem1):
    sems = (sem0, sem1)
    wid = lax.axis_index("s") * _NC + lax.axis_index("c")
    row0 = wid * (_GR * _NG)
    lanes = lax.iota(jnp.int32, _L)
    fzero = jnp.zeros((_L,), jnp.float32)
    c0vec = jnp.where(lanes == 0, jnp.full((_L,), _C0, jnp.float32), fzero)

    pltpu.sync_copy(tgt_hbm.at[pl.ds(row0, _GR * _NG)], t_v)
    hit_v[...] = fzero

    def process_chunk(b, coff, width, nvec, ts):
        for r in range(_GR):
            t_r = ts[r]

            @pl.when(t_r != 0)
            def _(r=r):
                s = _row_sum(x_v, b, r, nvec)
                plsc.addupdate(acc_v.at[r], s)

            @pl.when((t_r != 0) & (t_r >= coff) & (t_r < coff + width))
            def _(r=r, t_r=t_r):
                tr = t_r - coff
                ta = pl.multiple_of((tr >> 4) << 4, _L)
                v = x_v[b, r, pl.ds(ta, _L)]
                plsc.addupdate(
                    hit_v.at[:],
                    jnp.where(lanes == (tr - ta), -(_CONF - _S) * v, fzero))

    def group_body(g, total):
        gr0 = row0 + g * _GR
        t16 = t_v[pl.ds(pl.multiple_of(g * _GR, _L), _L)]
        ts = [t16[r] for r in range(_GR)]

        for r in range(_GR):
            acc_v[r] = fzero

        pltpu.async_copy(x_hbm.at[pl.ds(gr0, _GR), pl.ds(0, _CW)],
                         x_v.at[0], sems[0])
        pltpu.async_copy(x_hbm.at[pl.ds(gr0, _GR), pl.ds(_CW, _CW)],
                         x_v.at[1], sems[1])

        def pair(p, carry, gr0=gr0, ts=ts):
            for b in range(2):
                cj = p * 2 + b
                pltpu.make_async_copy(
                    x_hbm.at[pl.ds(gr0, _GR), pl.ds(cj * _CW, _CW)],
                    x_v.at[b], sems[b]).wait()
                process_chunk(b, cj * _CW, _CW, _CW // _L, ts)

                @pl.when(cj == 0)
                def _():
                    for r in range(_GR):
                        @pl.when(ts[r] != 0)
                        def _(r=r):
                            v0 = x_v[0, r, pl.ds(0, _L)]
                            val = jnp.where(lanes == 0, _S * v0, fzero) + c0vec
                            plsc.addupdate(hit_v.at[:], val)

                nxt = cj + 2
                @pl.when(nxt < _NFULL)
                def _():
                    pltpu.async_copy(
                        x_hbm.at[pl.ds(gr0, _GR), pl.ds(nxt * _CW, _CW)],
                        x_v.at[b], sems[b])
            return carry

        lax.fori_loop(0, _NFULL // 2, pair, 0)

        pltpu.async_copy(
            x_hbm.at[pl.ds(gr0, _GR), pl.ds(_NFULL * _CW, _TAILW)],
            x_v.at[0].at[:, pl.ds(0, _TAILW)], sems[0]).wait()
        process_chunk(0, _NFULL * _CW, _TAILW, _TAILW // _L, ts)

        dense = fzero
        for r in range(_GR):
            dense = dense + acc_v[r]
        return total + (-_S) * dense

    total = lax.fori_loop(0, _NG, group_body, fzero)

    stage_v[...] = total + hit_v[...]
    pltpu.sync_copy(stage_v, out_hbm.at[wid])


def _make_sc():
    return pl.kernel(
        _sc_body,
        mesh=plsc.VectorSubcoreMesh(core_axis_name="c", subcore_axis_name="s"),
        out_type=jax.ShapeDtypeStruct((_NW, _L), jnp.float32),
        scratch_types=[
            pltpu.VMEM((2, _GR, _CW), jnp.float32),
            pltpu.VMEM((_GR * _NG,), jnp.int32),
            pltpu.VMEM((_GR, _L), jnp.float32),
            pltpu.VMEM((_L,), jnp.float32),
            pltpu.VMEM((_L,), jnp.float32),
            pltpu.SemaphoreType.DMA,
            pltpu.SemaphoreType.DMA,
        ],
        compiler_params=pltpu.CompilerParams(use_tc_tiling_on_sc=True),
    )


def _tc_tail_body(x_ref, t_ref, out_ref):
    t = t_ref[...]
    rowmask = t != 0
    xb = x_ref[...]
    cols = lax.broadcasted_iota(jnp.int32, (_N, 128), 1) + _SC_COLS
    valid = rowmask & (cols < _SIZE)
    hit = (cols == t) & rowmask
    out_ref[0, 0] = (-_S * jnp.sum(jnp.where(valid, xb, 0.0))
                     - (_CONF - _S) * jnp.sum(jnp.where(hit, xb, 0.0)))


_tc_tail = pl.pallas_call(
    _tc_tail_body,
    grid=(1,),
    in_specs=[
        pl.BlockSpec((_N, 128), lambda i: (0, _SC_COLS // 128)),
        pl.BlockSpec((_N, 1), lambda i: (0, 0)),
    ],
    out_specs=pl.BlockSpec((1, 1), lambda i: (0, 0), memory_space=pltpu.SMEM),
    out_shape=jax.ShapeDtypeStruct((1, 1), jnp.float32),
)


def kernel(x, target):
    tgt = target.astype(jnp.int32)
    partials = _make_sc()(x, tgt)
    tail = _tc_tail(x, tgt.reshape(_N, 1))
    return jnp.sum(partials) + tail[0, 0]


# speedup vs baseline: 1.1198x; 1.1198x over previous
import math

import jax
import jax.numpy as jnp
from jax import lax
from jax.experimental import pallas as pl
from jax.experimental.pallas import tpu as pltpu
from jax.experimental.pallas import tpu_sc as plsc

_SIZE = 100000
_N = 2048
_SMOOTHING = 0.1
_CONF = 1.0 - _SMOOTHING
_S = _SMOOTHING / (_SIZE - 2)
_C0 = (_SIZE - 2) * _S * math.log(_S) + _CONF * math.log(_CONF)

_NC = 2
_NW = 32
_GR = 16
_SC_ROWS = 1024
_NG = _SC_ROWS // _NW // _GR
_CW = 2048
_NFULL = 48
_TAILW = 1664
_SC_COLS = _NFULL * _CW + _TAILW
_L = 16


def _row_sum(x_v, b, r, nvec):
    def step(k, s, r=r, b=b):
        base = k * 8 * _L
        t0 = x_v[b, r, pl.ds(base, _L)] + x_v[b, r, pl.ds(base + _L, _L)]
        t1 = (x_v[b, r, pl.ds(base + 2 * _L, _L)]
              + x_v[b, r, pl.ds(base + 3 * _L, _L)])
        t2 = (x_v[b, r, pl.ds(base + 4 * _L, _L)]
              + x_v[b, r, pl.ds(base + 5 * _L, _L)])
        t3 = (x_v[b, r, pl.ds(base + 6 * _L, _L)]
              + x_v[b, r, pl.ds(base + 7 * _L, _L)])
        return s + ((t0 + t1) + (t2 + t3))
    return lax.fori_loop(0, nvec // 8, step, jnp.zeros((_L,), jnp.float32))


def _sc_body(x_hbm, tgt_hbm, out_hbm, x_v, t_v, acc_v, hit_v, stage_v,
             sem0, sem1):
    sems = (sem0, sem1)
    wid = lax.axis_index("s") * _NC + lax.axis_index("c")
    row0 = wid * (_GR * _NG)
    lanes = lax.iota(jnp.int32, _L)
    fzero = jnp.zeros((_L,), jnp.float32)
    c0vec = jnp.where(lanes == 0, jnp.full((_L,), _C0, jnp.float32), fzero)

    pltpu.sync_copy(tgt_hbm.at[pl.ds(row0, _GR * _NG)], t_v)
    hit_v[...] = fzero

    def process_chunk(b, coff, width, nvec, ts):
        for r in range(_GR):
            t_r = ts[r]

            @pl.when(t_r != 0)
            def _(r=r):
                s = _row_sum(x_v, b, r, nvec)
                plsc.addupdate(acc_v.at[r], s)

            @pl.when((t_r != 0) & (t_r >= coff) & (t_r < coff + width))
            def _(r=r, t_r=t_r):
                tr = t_r - coff
                ta = pl.multiple_of((tr >> 4) << 4, _L)
                v = x_v[b, r, pl.ds(ta, _L)]
                plsc.addupdate(
                    hit_v.at[:],
                    jnp.where(lanes == (tr - ta), -(_CONF - _S) * v, fzero))

    def group_body(g, total):
        gr0 = row0 + g * _GR
        t16 = t_v[pl.ds(pl.multiple_of(g * _GR, _L), _L)]
        ts = [t16[r] for r in range(_GR)]

        for r in range(_GR):
            acc_v[r] = fzero

        pltpu.async_copy(x_hbm.at[pl.ds(gr0, _GR), pl.ds(0, _CW)],
                         x_v.at[0], sems[0])
        pltpu.async_copy(x_hbm.at[pl.ds(gr0, _GR), pl.ds(_CW, _CW)],
                         x_v.at[1], sems[1])

        def pair(p, carry, gr0=gr0, ts=ts):
            for b in range(2):
                cj = p * 2 + b
                pltpu.make_async_copy(
                    x_hbm.at[pl.ds(gr0, _GR), pl.ds(cj * _CW, _CW)],
                    x_v.at[b], sems[b]).wait()
                process_chunk(b, cj * _CW, _CW, _CW // _L, ts)

                @pl.when(cj == 0)
                def _():
                    for r in range(_GR):
                        @pl.when(ts[r] != 0)
                        def _(r=r):
                            v0 = x_v[0, r, pl.ds(0, _L)]
                            val = jnp.where(lanes == 0, _S * v0, fzero) + c0vec
                            plsc.addupdate(hit_v.at[:], val)

                nxt = cj + 2
                @pl.when(nxt < _NFULL)
                def _():
                    pltpu.async_copy(
                        x_hbm.at[pl.ds(gr0, _GR), pl.ds(nxt * _CW, _CW)],
                        x_v.at[b], sems[b])
            return carry

        lax.fori_loop(0, _NFULL // 2, pair, 0)

        pltpu.async_copy(
            x_hbm.at[pl.ds(gr0, _GR), pl.ds(_NFULL * _CW, _TAILW)],
            x_v.at[0].at[:, pl.ds(0, _TAILW)], sems[0]).wait()
        process_chunk(0, _NFULL * _CW, _TAILW, _TAILW // _L, ts)

        dense = fzero
        for r in range(_GR):
            dense = dense + acc_v[r]
        return total + (-_S) * dense

    total = lax.fori_loop(0, _NG, group_body, fzero)

    stage_v[...] = total + hit_v[...]
    pltpu.sync_copy(stage_v, out_hbm.at[wid])


def _make_sc():
    return pl.kernel(
        _sc_body,
        mesh=plsc.VectorSubcoreMesh(core_axis_name="c", subcore_axis_name="s"),
        out_type=jax.ShapeDtypeStruct((_NW, _L), jnp.float32),
        scratch_types=[
            pltpu.VMEM((2, _GR, _CW), jnp.float32),
            pltpu.VMEM((_GR * _NG,), jnp.int32),
            pltpu.VMEM((_GR, _L), jnp.float32),
            pltpu.VMEM((_L,), jnp.float32),
            pltpu.VMEM((_L,), jnp.float32),
            pltpu.SemaphoreType.DMA,
            pltpu.SemaphoreType.DMA,
        ],
        compiler_params=pltpu.CompilerParams(use_tc_tiling_on_sc=True),
    )


def _tc_tail_body(x_ref, t_ref, out_ref):
    t = t_ref[...]
    rowmask = t != 0
    xb = x_ref[...]
    cols = lax.broadcasted_iota(jnp.int32, (_SC_ROWS, 128), 1) + _SC_COLS
    valid = rowmask & (cols < _SIZE)
    hit = (cols == t) & rowmask
    out_ref[0, 0] = (-_S * jnp.sum(jnp.where(valid, xb, 0.0))
                     - (_CONF - _S) * jnp.sum(jnp.where(hit, xb, 0.0)))


_tc_tail = pl.pallas_call(
    _tc_tail_body,
    grid=(1,),
    in_specs=[
        pl.BlockSpec((_SC_ROWS, 128), lambda i: (0, _SC_COLS // 128)),
        pl.BlockSpec((_SC_ROWS, 1), lambda i: (0, 0)),
    ],
    out_specs=pl.BlockSpec((1, 1), lambda i: (0, 0), memory_space=pltpu.SMEM),
    out_shape=jax.ShapeDtypeStruct((1, 1), jnp.float32),
)

_BR = 512
_BC = 2048
_NCB = (_SIZE + _BC - 1) // _BC
_ROW_OFF = _SC_ROWS // _BR


def _tc_main_body(x_ref, t_ref, out_ref):
    i = pl.program_id(0)
    j = pl.program_id(1)

    @pl.when(jnp.logical_and(i == 0, j == 0))
    def _init():
        out_ref[0, 0] = 0.0

    t = t_ref[...]
    rowmask = t != 0
    xb = x_ref[...]
    cols = lax.broadcasted_iota(jnp.int32, (_BR, _BC), 1) + j * _BC

    hit = (cols == t) & rowmask
    out_ref[0, 0] += -(_CONF - _S) * jnp.sum(jnp.where(hit, xb, 0.0))

    valid = rowmask & (cols < _SIZE)
    out_ref[0, 0] += -_S * jnp.sum(jnp.where(valid, xb, 0.0))

    @pl.when(j == 0)
    def _row_terms():
        x0 = xb[:, 0:1]
        out_ref[0, 0] += _S * jnp.sum(jnp.where(rowmask, x0, 0.0))
        out_ref[0, 0] += _C0 * jnp.sum(jnp.where(rowmask, 1.0, 0.0))


_tc_main = pl.pallas_call(
    _tc_main_body,
    grid=((_N - _SC_ROWS) // _BR, _NCB),
    in_specs=[
        pl.BlockSpec((_BR, _BC), lambda i, j: (i + _ROW_OFF, j)),
        pl.BlockSpec((_BR, 1), lambda i, j: (i + _ROW_OFF, 0)),
    ],
    out_specs=pl.BlockSpec((1, 1), lambda i, j: (0, 0), memory_space=pltpu.SMEM),
    out_shape=jax.ShapeDtypeStruct((1, 1), jnp.float32),
    compiler_params=pltpu.CompilerParams(
        dimension_semantics=("arbitrary", "arbitrary"),
    ),
)


def kernel(x, target):
    tgt = target.astype(jnp.int32)
    partials = _make_sc()(x, tgt)
    t2 = tgt.reshape(_N, 1)
    main = _tc_main(x, t2)
    tail = _tc_tail(x, t2)
    return jnp.sum(partials) + main[0, 0] + tail[0, 0]
